# Initial kernel scaffold; baseline (speedup 1.0000x reference)
import functools

import jax
import jax.numpy as jnp
from jax import lax
from jax.experimental import pallas as pl
from jax.experimental.pallas import tpu as pltpu
from jax.experimental.pallas import tpu_sc as plsc

N = 100000
D = 128
B = 4096
NC = 2
NS = 16
CH = 128
NCHUNK = N // CH
TAIL = N - NCHUNK * CH
ITERS = (NCHUNK + NS - 1) // NS
SEG_PER_TILE = B // NS
CW = 16


def _sc_pool_body(xq, xa, xc, xt, bq, ba, bc, bt,
                  sums_out, cnts_out,
                  rows_v, idx_v, idx_tail_v, ones_v, zb_v, zc_v,
                  acc0, acc1, cnt0, cnt1):
    c = lax.axis_index("c")
    s = lax.axis_index("s")

    zeros16 = jnp.zeros((16,), jnp.float32)
    ones16 = jnp.ones((16,), jnp.float32)

    def fill_zb(i, _):
        for k in range(D // 16):
            zb_v[i, pl.ds(k * 16, 16)] = zeros16
        zc_v[i, pl.ds(0, 16)] = zeros16
        return 0
    lax.fori_loop(0, SEG_PER_TILE, fill_zb, 0)

    def fill_ones(i, _):
        ones_v[i, pl.ds(0, 16)] = ones16
        return 0
    lax.fori_loop(0, CH, fill_ones, 0)

    seg_base = s * SEG_PER_TILE
    for acc, cnt in ((acc0, cnt0), (acc1, cnt1)):
        pltpu.sync_copy(zb_v, acc.at[pl.ds(seg_base, SEG_PER_TILE), :])
        pltpu.sync_copy(zc_v, cnt.at[pl.ds(seg_base, SEG_PER_TILE), :])
    plsc.subcore_barrier()

    def do_chunk(x_hbm, b_hbm, acc, cnt, base, rows):
        pltpu.sync_copy(x_hbm.at[pl.ds(base, rows), :], rows_v.at[pl.ds(0, rows), :])
        if rows == CH:
            pltpu.sync_copy(b_hbm.at[pl.ds(base, rows)], idx_v)
            pltpu.sync_copy(rows_v, acc.at[idx_v], add=True)
            pltpu.sync_copy(ones_v, cnt.at[idx_v], add=True)
        else:
            pltpu.sync_copy(b_hbm.at[pl.ds(base, rows)], idx_tail_v)
            pltpu.sync_copy(rows_v.at[pl.ds(0, rows), :], acc.at[idx_tail_v], add=True)
            pltpu.sync_copy(ones_v.at[pl.ds(0, rows), :], cnt.at[idx_tail_v], add=True)

    for t, (xA, bA, xB, bB) in ((0, (xq, bq, xc, bc)), (1, (xa, ba, xt, bt))):
        acc, cnt = (acc0, cnt0) if t == 0 else (acc1, cnt1)

        def body(j, _):
            chunk = s + j * NS

            @pl.when(chunk < NCHUNK)
            def _():
                base = chunk * CH

                @pl.when(c == 0)
                def _():
                    do_chunk(xA, bA, acc, cnt, base, CH)

                @pl.when(c == 1)
                def _():
                    do_chunk(xB, bB, acc, cnt, base, CH)
            return 0
        lax.fori_loop(0, ITERS, body, 0)

        if TAIL:
            @pl.when(s == 0)
            def _():
                @pl.when(c == 0)
                def _():
                    do_chunk(xA, bA, acc, cnt, NCHUNK * CH, TAIL)

                @pl.when(c == 1)
                def _():
                    do_chunk(xB, bB, acc, cnt, NCHUNK * CH, TAIL)

    plsc.subcore_barrier()

    for t, (acc, cnt) in ((0, (acc0, cnt0)), (1, (acc1, cnt1))):
        ti = 2 * c + t
        pltpu.sync_copy(acc.at[pl.ds(seg_base, SEG_PER_TILE), :], zb_v)
        pltpu.sync_copy(zb_v, sums_out.at[ti, pl.ds(seg_base, SEG_PER_TILE), :])
        pltpu.sync_copy(cnt.at[pl.ds(seg_base, SEG_PER_TILE), :], zc_v)
        pltpu.sync_copy(zc_v, cnts_out.at[ti, pl.ds(seg_base, SEG_PER_TILE), :])


_sc_pool = functools.partial(
    pl.kernel,
    out_type=(
        jax.ShapeDtypeStruct((4, B, D), jnp.float32),
        jax.ShapeDtypeStruct((4, B, CW), jnp.float32),
    ),
    mesh=plsc.VectorSubcoreMesh(core_axis_name="c", subcore_axis_name="s",
                                num_cores=NC, num_subcores=NS),
    scratch_types=[
        pltpu.VMEM((CH, D), jnp.float32),
        pltpu.VMEM((CH,), jnp.int32),
        pltpu.VMEM((TAIL,), jnp.int32),
        pltpu.VMEM((CH, CW), jnp.float32),
        pltpu.VMEM((SEG_PER_TILE, D), jnp.float32),
        pltpu.VMEM((SEG_PER_TILE, CW), jnp.float32),
        pltpu.MemorySpace.VMEM_SHARED((B, D), jnp.float32),
        pltpu.MemorySpace.VMEM_SHARED((B, D), jnp.float32),
        pltpu.MemorySpace.VMEM_SHARED((B, CW), jnp.float32),
        pltpu.MemorySpace.VMEM_SHARED((B, CW), jnp.float32),
    ],
)(_sc_pool_body)


def _mlp_body(sums_ref, cnts_ref, post_ref, w1_ref, b1_ref, w2_ref, b2_ref, o_ref):
    parts = []
    for t in range(4):
        cnt = jnp.maximum(cnts_ref[t][:, 0:1], 1.0)
        parts.append(sums_ref[t] / cnt)
    parts.append(post_ref[...])
    x = jnp.concatenate(parts, axis=1)
    h = jnp.dot(x, w1_ref[...], preferred_element_type=jnp.float32) + b1_ref[...]
    h = jnp.where(h >= 0, h, 0.01 * h)
    o = jnp.dot(h, w2_ref[...], preferred_element_type=jnp.float32) + b2_ref[...]
    o = jnp.where(o >= 0, o, 0.01 * o)
    m = jnp.max(o, axis=-1, keepdims=True)
    e = jnp.exp(o - m)
    o_ref[...] = e / jnp.sum(e, axis=-1, keepdims=True)


def _mlp(sums, cnts, post_emb, W1, b1, W2, b2):
    BM = 256
    grid = (B // BM,)
    return pl.pallas_call(
        _mlp_body,
        grid=grid,
        in_specs=[
            pl.BlockSpec((4, BM, D), lambda i: (0, i, 0)),
            pl.BlockSpec((4, BM, CW), lambda i: (0, i, 0)),
            pl.BlockSpec((BM, post_emb.shape[1]), lambda i: (i, 0)),
            pl.BlockSpec(W1.shape, lambda i: (0, 0)),
            pl.BlockSpec((1, b1.shape[0]), lambda i: (0, 0)),
            pl.BlockSpec(W2.shape, lambda i: (0, 0)),
            pl.BlockSpec((1, b2.shape[0]), lambda i: (0, 0)),
        ],
        out_specs=pl.BlockSpec((BM, 2), lambda i: (i, 0)),
        out_shape=jax.ShapeDtypeStruct((B, 2), jnp.float32),
    )(sums, cnts, post_emb, W1, b1.reshape(1, -1), W2, b2.reshape(1, -1))


def kernel(x_question, x_answer, x_comment, x_tag,
           batch_question, batch_answer, batch_comment, batch_tag,
           edge_index, post_emb, W1, b1, W2, b2):
    sums, cnts = _sc_pool(x_question, x_answer, x_comment, x_tag,
                          batch_question, batch_answer, batch_comment, batch_tag)
    return _mlp(sums, cnts, post_emb, W1, b1, W2, b2)


# speedup vs baseline: 3.3144x; 3.3144x over previous
import functools

import jax
import jax.numpy as jnp
from jax import lax
from jax.experimental import pallas as pl
from jax.experimental.pallas import tpu as pltpu
from jax.experimental.pallas import tpu_sc as plsc

N = 100000
D = 128
B = 4096
NC = 2
NS = 16
NW = NC * NS
CH = 128
NCHUNK = N // CH
TAIL = N - NCHUNK * CH
ITERS = (NCHUNK + NW - 1) // NW
SEG_PER_TILE = B // NS


def _sc_pool_body(xq, xa, xc, xt, bq, ba, bc, bt,
                  sums_out, cnts_out,
                  rows_v, idx_v, idx_tail_v, ones_v, zb_v,
                  acc):
    c = lax.axis_index("c")
    s = lax.axis_index("s")
    wid = c * NS + s

    zeros16 = jnp.zeros((16,), jnp.float32)
    ones16 = jnp.ones((16,), jnp.float32)

    def fill_zb(i, _):
        for k in range(D // 16):
            zb_v[i, pl.ds(k * 16, 16)] = zeros16
        return 0
    lax.fori_loop(0, SEG_PER_TILE, fill_zb, 0)

    def fill_ones(i, _):
        for k in range(D // 16):
            ones_v[i, pl.ds(k * 16, 16)] = ones16
        return 0
    lax.fori_loop(0, CH, fill_ones, 0)

    seg_base = s * SEG_PER_TILE

    def zero_acc():
        pltpu.sync_copy(zb_v, acc.at[pl.ds(seg_base, SEG_PER_TILE), :])
        plsc.subcore_barrier()

    def flush_acc(out, ti):
        plsc.subcore_barrier()
        pltpu.sync_copy(acc.at[pl.ds(seg_base, SEG_PER_TILE), :], zb_v)
        pltpu.sync_copy(zb_v, out.at[c, ti, pl.ds(seg_base, SEG_PER_TILE), :])
        lax.fori_loop(0, SEG_PER_TILE, fill_zb, 0)

    def data_chunk(x_hbm, b_hbm, base, rows):
        pltpu.sync_copy(x_hbm.at[pl.ds(base, rows), :], rows_v.at[pl.ds(0, rows), :])
        if rows == CH:
            pltpu.sync_copy(b_hbm.at[pl.ds(base, rows)], idx_v)
            pltpu.sync_copy(rows_v, acc.at[idx_v], add=True)
        else:
            pltpu.sync_copy(b_hbm.at[pl.ds(base, rows)], idx_tail_v)
            pltpu.sync_copy(rows_v.at[pl.ds(0, rows), :], acc.at[idx_tail_v], add=True)

    def ones_chunk(b_hbm, base, rows):
        if rows == CH:
            pltpu.sync_copy(b_hbm.at[pl.ds(base, rows)], idx_v)
            pltpu.sync_copy(ones_v, acc.at[idx_v], add=True)
        else:
            pltpu.sync_copy(b_hbm.at[pl.ds(base, rows)], idx_tail_v)
            pltpu.sync_copy(ones_v.at[pl.ds(0, rows), :], acc.at[idx_tail_v], add=True)

    def scatter_pass(do_chunk):
        def body(j, _):
            chunk = wid + j * NW

            @pl.when(chunk < NCHUNK)
            def _():
                do_chunk(chunk * CH, CH)
            return 0
        lax.fori_loop(0, ITERS, body, 0)

        if TAIL:
            @pl.when(wid == 0)
            def _():
                do_chunk(NCHUNK * CH, TAIL)

    for ti, (x_hbm, b_hbm) in enumerate(((xq, bq), (xa, ba), (xc, bc), (xt, bt))):
        zero_acc()
        scatter_pass(lambda base, rows: data_chunk(x_hbm, b_hbm, base, rows))
        flush_acc(sums_out, ti)

        zero_acc()
        scatter_pass(lambda base, rows: ones_chunk(b_hbm, base, rows))
        flush_acc(cnts_out, ti)


_sc_pool = functools.partial(
    pl.kernel,
    out_type=(
        jax.ShapeDtypeStruct((NC, 4, B, D), jnp.float32),
        jax.ShapeDtypeStruct((NC, 4, B, D), jnp.float32),
    ),
    mesh=plsc.VectorSubcoreMesh(core_axis_name="c", subcore_axis_name="s",
                                num_cores=NC, num_subcores=NS),
    scratch_types=[
        pltpu.VMEM((CH, D), jnp.float32),
        pltpu.VMEM((CH,), jnp.int32),
        pltpu.VMEM((TAIL,), jnp.int32),
        pltpu.VMEM((CH, D), jnp.float32),
        pltpu.VMEM((SEG_PER_TILE, D), jnp.float32),
        pltpu.MemorySpace.VMEM_SHARED((B, D), jnp.float32),
    ],
)(_sc_pool_body)


def _mlp_body(sums_ref, cnts_ref, post_ref, w1_ref, b1_ref, w2_ref, b2_ref, o_ref):
    parts = []
    for t in range(4):
        s = sums_ref[0, t] + sums_ref[1, t]
        cnt = cnts_ref[0, t][:, 0:1] + cnts_ref[1, t][:, 0:1]
        parts.append(s / jnp.maximum(cnt, 1.0))
    parts.append(post_ref[...])
    x = jnp.concatenate(parts, axis=1)
    h = jnp.dot(x, w1_ref[...], preferred_element_type=jnp.float32) + b1_ref[...]
    h = jnp.where(h >= 0, h, 0.01 * h)
    o = jnp.dot(h, w2_ref[...], preferred_element_type=jnp.float32) + b2_ref[...]
    o = jnp.where(o >= 0, o, 0.01 * o)
    m = jnp.max(o, axis=-1, keepdims=True)
    e = jnp.exp(o - m)
    o_ref[...] = e / jnp.sum(e, axis=-1, keepdims=True)


def _mlp(sums, cnts, post_emb, W1, b1, W2, b2):
    BM = 256
    grid = (B // BM,)
    return pl.pallas_call(
        _mlp_body,
        grid=grid,
        in_specs=[
            pl.BlockSpec((NC, 4, BM, D), lambda i: (0, 0, i, 0)),
            pl.BlockSpec((NC, 4, BM, D), lambda i: (0, 0, i, 0)),
            pl.BlockSpec((BM, post_emb.shape[1]), lambda i: (i, 0)),
            pl.BlockSpec(W1.shape, lambda i: (0, 0)),
            pl.BlockSpec((1, b1.shape[0]), lambda i: (0, 0)),
            pl.BlockSpec(W2.shape, lambda i: (0, 0)),
            pl.BlockSpec((1, b2.shape[0]), lambda i: (0, 0)),
        ],
        out_specs=pl.BlockSpec((BM, 2), lambda i: (i, 0)),
        out_shape=jax.ShapeDtypeStruct((B, 2), jnp.float32),
    )(sums, cnts, post_emb, W1, b1.reshape(1, -1), W2, b2.reshape(1, -1))


def kernel(x_question, x_answer, x_comment, x_tag,
           batch_question, batch_answer, batch_comment, batch_tag,
           edge_index, post_emb, W1, b1, W2, b2):
    sums, cnts = _sc_pool(x_question, x_answer, x_comment, x_tag,
                          batch_question, batch_answer, batch_comment, batch_tag)
    return _mlp(sums, cnts, post_emb, W1, b1, W2, b2)


# speedup vs baseline: 5.1226x; 1.5456x over previous
import functools

import jax
import jax.numpy as jnp
from jax import lax
from jax.experimental import pallas as pl
from jax.experimental.pallas import tpu as pltpu
from jax.experimental.pallas import tpu_sc as plsc

N = 100000
D = 128
B = 4096
NC = 2
NS = 16
NW = NC * NS
CH = 128
NCHUNK = N // CH
TAIL = N - NCHUNK * CH
ITERS = (NCHUNK + NW - 1) // NW
HITERS = (ITERS + 1) // 2
SEG_PER_TILE = B // NS


def _sc_pool_body(xq, xa, xc, xt, bq, ba, bc, bt,
                  sums_out, cnts_out,
                  rows2_v, idx2_v, idx_tail_v, ones_v, zb_v,
                  acc, sem_r0, sem_r1, sem_i0, sem_i1):
    c = lax.axis_index("c")
    s = lax.axis_index("s")
    wid = c * NS + s

    sem_r = (sem_r0, sem_r1)
    sem_i = (sem_i0, sem_i1)

    zeros16 = jnp.zeros((16,), jnp.float32)
    ones16 = jnp.ones((16,), jnp.float32)

    def fill_zb(i, _):
        for k in range(D // 16):
            zb_v[i, pl.ds(k * 16, 16)] = zeros16
        return 0
    lax.fori_loop(0, SEG_PER_TILE, fill_zb, 0)

    def fill_ones(i, _):
        for k in range(D // 16):
            ones_v[i, pl.ds(k * 16, 16)] = ones16
        return 0
    lax.fori_loop(0, CH, fill_ones, 0)

    seg_base = s * SEG_PER_TILE

    def zero_acc():
        pltpu.sync_copy(zb_v, acc.at[pl.ds(seg_base, SEG_PER_TILE), :])
        plsc.subcore_barrier()

    def flush_acc(out, ti):
        plsc.subcore_barrier()
        pltpu.sync_copy(acc.at[pl.ds(seg_base, SEG_PER_TILE), :], zb_v)
        pltpu.sync_copy(zb_v, out.at[c, ti, pl.ds(seg_base, SEG_PER_TILE), :])
        lax.fori_loop(0, SEG_PER_TILE, fill_zb, 0)

    def rows_cp(x_hbm, q, b):
        g = wid + q * NW
        return pltpu.make_async_copy(
            x_hbm.at[pl.ds(g * CH, CH), :], rows2_v.at[b], sem_r[b])

    def idx_cp(b_hbm, q, b):
        g = wid + q * NW
        return pltpu.make_async_copy(
            b_hbm.at[pl.ds(g * CH, CH)], idx2_v.at[b], sem_i[b])

    def in_bounds(q):
        return (wid + q * NW) < NCHUNK

    def data_pass(x_hbm, b_hbm):
        for b in range(2):
            @pl.when(in_bounds(b))
            def _():
                rows_cp(x_hbm, b, b).start()
                idx_cp(b_hbm, b, b).start()

        def body(j2, _):
            for b in range(2):
                q = 2 * j2 + b

                @pl.when(in_bounds(q))
                def _():
                    rows_cp(x_hbm, q, b).wait()
                    idx_cp(b_hbm, q, b).wait()
                    pltpu.sync_copy(rows2_v.at[b], acc.at[idx2_v.at[b]], add=True)

                    @pl.when(in_bounds(q + 2))
                    def _():
                        rows_cp(x_hbm, q + 2, b).start()
                        idx_cp(b_hbm, q + 2, b).start()
            return 0
        lax.fori_loop(0, HITERS, body, 0)

        if TAIL:
            @pl.when(wid == 0)
            def _():
                pltpu.sync_copy(x_hbm.at[pl.ds(NCHUNK * CH, TAIL), :],
                                rows2_v.at[0, pl.ds(0, TAIL), :])
                pltpu.sync_copy(b_hbm.at[pl.ds(NCHUNK * CH, TAIL)], idx_tail_v)
                pltpu.sync_copy(rows2_v.at[0, pl.ds(0, TAIL), :],
                                acc.at[idx_tail_v], add=True)

    def ones_pass(b_hbm):
        for b in range(2):
            @pl.when(in_bounds(b))
            def _():
                idx_cp(b_hbm, b, b).start()

        def body(j2, _):
            for b in range(2):
                q = 2 * j2 + b

                @pl.when(in_bounds(q))
                def _():
                    idx_cp(b_hbm, q, b).wait()
                    pltpu.sync_copy(ones_v, acc.at[idx2_v.at[b]], add=True)

                    @pl.when(in_bounds(q + 2))
                    def _():
                        idx_cp(b_hbm, q + 2, b).start()
            return 0
        lax.fori_loop(0, HITERS, body, 0)

        if TAIL:
            @pl.when(wid == 0)
            def _():
                pltpu.sync_copy(b_hbm.at[pl.ds(NCHUNK * CH, TAIL)], idx_tail_v)
                pltpu.sync_copy(ones_v.at[pl.ds(0, TAIL), :],
                                acc.at[idx_tail_v], add=True)

    for ti, (x_hbm, b_hbm) in enumerate(((xq, bq), (xa, ba), (xc, bc), (xt, bt))):
        zero_acc()
        data_pass(x_hbm, b_hbm)
        flush_acc(sums_out, ti)

        zero_acc()
        ones_pass(b_hbm)
        flush_acc(cnts_out, ti)


_sc_pool = functools.partial(
    pl.kernel,
    out_type=(
        jax.ShapeDtypeStruct((NC, 4, B, D), jnp.float32),
        jax.ShapeDtypeStruct((NC, 4, B, D), jnp.float32),
    ),
    mesh=plsc.VectorSubcoreMesh(core_axis_name="c", subcore_axis_name="s",
                                num_cores=NC, num_subcores=NS),
    scratch_types=[
        pltpu.VMEM((2, CH, D), jnp.float32),
        pltpu.VMEM((2, CH), jnp.int32),
        pltpu.VMEM((TAIL,), jnp.int32),
        pltpu.VMEM((CH, D), jnp.float32),
        pltpu.VMEM((SEG_PER_TILE, D), jnp.float32),
        pltpu.MemorySpace.VMEM_SHARED((B, D), jnp.float32),
        pltpu.SemaphoreType.DMA,
        pltpu.SemaphoreType.DMA,
        pltpu.SemaphoreType.DMA,
        pltpu.SemaphoreType.DMA,
    ],
)(_sc_pool_body)


def _mlp_body(sums_ref, cnts_ref, post_ref, w1_ref, b1_ref, w2_ref, b2_ref, o_ref):
    parts = []
    for t in range(4):
        st = sums_ref[0, t] + sums_ref[1, t]
        cnt = cnts_ref[0, t][:, 0:1] + cnts_ref[1, t][:, 0:1]
        parts.append(st / jnp.maximum(cnt, 1.0))
    parts.append(post_ref[...])
    x = jnp.concatenate(parts, axis=1)
    h = jnp.dot(x, w1_ref[...], preferred_element_type=jnp.float32) + b1_ref[...]
    h = jnp.where(h >= 0, h, 0.01 * h)
    o = jnp.dot(h, w2_ref[...], preferred_element_type=jnp.float32) + b2_ref[...]
    o = jnp.where(o >= 0, o, 0.01 * o)
    m = jnp.max(o, axis=-1, keepdims=True)
    e = jnp.exp(o - m)
    o_ref[...] = e / jnp.sum(e, axis=-1, keepdims=True)


def _mlp(sums, cnts, post_emb, W1, b1, W2, b2):
    BM = 256
    grid = (B // BM,)
    return pl.pallas_call(
        _mlp_body,
        grid=grid,
        in_specs=[
            pl.BlockSpec((NC, 4, BM, D), lambda i: (0, 0, i, 0)),
            pl.BlockSpec((NC, 4, BM, D), lambda i: (0, 0, i, 0)),
            pl.BlockSpec((BM, post_emb.shape[1]), lambda i: (i, 0)),
            pl.BlockSpec(W1.shape, lambda i: (0, 0)),
            pl.BlockSpec((1, b1.shape[0]), lambda i: (0, 0)),
            pl.BlockSpec(W2.shape, lambda i: (0, 0)),
            pl.BlockSpec((1, b2.shape[0]), lambda i: (0, 0)),
        ],
        out_specs=pl.BlockSpec((BM, 2), lambda i: (i, 0)),
        out_shape=jax.ShapeDtypeStruct((B, 2), jnp.float32),
    )(sums, cnts, post_emb, W1, b1.reshape(1, -1), W2, b2.reshape(1, -1))


def kernel(x_question, x_answer, x_comment, x_tag,
           batch_question, batch_answer, batch_comment, batch_tag,
           edge_index, post_emb, W1, b1, W2, b2):
    sums, cnts = _sc_pool(x_question, x_answer, x_comment, x_tag,
                          batch_question, batch_answer, batch_comment,
                          batch_tag)
    return _mlp(sums, cnts, post_emb, W1, b1, W2, b2)


# speedup vs baseline: 5.1315x; 1.0017x over previous
import functools

import jax
import jax.numpy as jnp
from jax import lax
from jax.experimental import pallas as pl
from jax.experimental.pallas import tpu as pltpu
from jax.experimental.pallas import tpu_sc as plsc

N = 100000
D = 128
B = 4096
NC = 2
NS = 16
NW = NC * NS
CH = 128
NB = 4
NCHUNK = N // CH
TAIL = N - NCHUNK * CH
ITERS = (NCHUNK + NW - 1) // NW
QITERS = (ITERS + 2 + NB - 1) // NB
SEG_PER_TILE = B // NS
HSEG = SEG_PER_TILE // 2


def _sc_pool_body(xq, xa, xc, xt, bq, ba, bc, bt,
                  sums_out, cnts_out,
                  rows4_v, idx4_v, idx_tail_v, zb_v, acc,
                  sem_r0, sem_r1, sem_r2, sem_r3,
                  sem_i0, sem_i1, sem_i2, sem_i3):
    c = lax.axis_index("c")
    s = lax.axis_index("s")
    wid = c * NS + s

    sem_r = (sem_r0, sem_r1, sem_r2, sem_r3)
    sem_i = (sem_i0, sem_i1, sem_i2, sem_i3)

    zeros16 = jnp.zeros((16,), jnp.float32)
    ones16 = jnp.ones((16,), jnp.float32)

    def fill_zb(i, _):
        for k in range(D // 16):
            zb_v[i, pl.ds(k * 16, 16)] = zeros16
        return 0
    lax.fori_loop(0, HSEG, fill_zb, 0)

    def fill_ones(i, _):
        for k in range(D // 16):
            rows4_v[0, i, pl.ds(k * 16, 16)] = ones16
        return 0

    seg_base = s * SEG_PER_TILE

    def zero_acc():
        for h in range(2):
            pltpu.sync_copy(zb_v, acc.at[pl.ds(seg_base + h * HSEG, HSEG), :])
        plsc.subcore_barrier()

    def flush_acc(out, ti):
        plsc.subcore_barrier()
        for h in range(2):
            pltpu.sync_copy(acc.at[pl.ds(seg_base + h * HSEG, HSEG), :], zb_v)
            pltpu.sync_copy(zb_v, out.at[c, ti,
                                         pl.ds(seg_base + h * HSEG, HSEG), :])
        lax.fori_loop(0, HSEG, fill_zb, 0)

    def rows_cp(x_hbm, q, b):
        g = wid + q * NW
        return pltpu.make_async_copy(
            x_hbm.at[pl.ds(g * CH, CH), :], rows4_v.at[b], sem_r[b])

    def idx_cp(b_hbm, q, b):
        g = wid + q * NW
        return pltpu.make_async_copy(
            b_hbm.at[pl.ds(g * CH, CH)], idx4_v.at[b], sem_i[b])


    def in_bounds(q):
        return (wid + q * NW) < NCHUNK

    def run_pass(x_hbm, b_hbm, with_rows):
        def prefetch(q, b):
            if with_rows:
                rows_cp(x_hbm, q, b).start()
            idx_cp(b_hbm, q, b).start()

        def fire(q, b):
            if with_rows:
                rows_cp(x_hbm, q, b).wait()
                idx_cp(b_hbm, q, b).wait()
                pltpu.sync_copy(rows4_v.at[b], acc.at[idx4_v.at[b]], add=True)
            else:
                idx_cp(b_hbm, q, b).wait()
                pltpu.sync_copy(rows4_v.at[0], acc.at[idx4_v.at[b]], add=True)

        for b in range(2):
            @pl.when(in_bounds(b))
            def _():
                prefetch(b, b)

        def body(j, _):
            for b in range(NB):
                q = j * NB + b

                @pl.when(in_bounds(q))
                def _():
                    fire(q, b)

                @pl.when(in_bounds(q + 2))
                def _():
                    prefetch(q + 2, (b + 2) % NB)
            return 0
        lax.fori_loop(0, QITERS, body, 0)

        if TAIL:
            @pl.when(wid == 0)
            def _():
                pltpu.sync_copy(b_hbm.at[pl.ds(NCHUNK * CH, TAIL)], idx_tail_v)
                if with_rows:
                    pltpu.sync_copy(x_hbm.at[pl.ds(NCHUNK * CH, TAIL), :],
                                    rows4_v.at[0, pl.ds(0, TAIL), :])
                    pltpu.sync_copy(rows4_v.at[0, pl.ds(0, TAIL), :],
                                    acc.at[idx_tail_v], add=True)
                else:
                    pltpu.sync_copy(rows4_v.at[0, pl.ds(0, TAIL), :],
                                    acc.at[idx_tail_v], add=True)

    for ti, (x_hbm, b_hbm) in enumerate(((xq, bq), (xa, ba), (xc, bc), (xt, bt))):
        zero_acc()
        run_pass(x_hbm, b_hbm, True)
        flush_acc(sums_out, ti)

        zero_acc()
        lax.fori_loop(0, CH, fill_ones, 0)
        run_pass(x_hbm, b_hbm, False)
        flush_acc(cnts_out, ti)


_sc_pool = functools.partial(
    pl.kernel,
    out_type=(
        jax.ShapeDtypeStruct((NC, 4, B, D), jnp.float32),
        jax.ShapeDtypeStruct((NC, 4, B, D), jnp.float32),
    ),
    mesh=plsc.VectorSubcoreMesh(core_axis_name="c", subcore_axis_name="s",
                                num_cores=NC, num_subcores=NS),
    scratch_types=[
        pltpu.VMEM((NB, CH, D), jnp.float32),
        pltpu.VMEM((NB, CH), jnp.int32),
        pltpu.VMEM((TAIL,), jnp.int32),
        pltpu.VMEM((HSEG, D), jnp.float32),
        pltpu.MemorySpace.VMEM_SHARED((B, D), jnp.float32),
        pltpu.SemaphoreType.DMA,
        pltpu.SemaphoreType.DMA,
        pltpu.SemaphoreType.DMA,
        pltpu.SemaphoreType.DMA,
        pltpu.SemaphoreType.DMA,
        pltpu.SemaphoreType.DMA,
        pltpu.SemaphoreType.DMA,
        pltpu.SemaphoreType.DMA,
    ],
)(_sc_pool_body)


def _mlp_body(sums_ref, cnts_ref, post_ref, w1_ref, b1_ref, w2_ref, b2_ref, o_ref):
    parts = []
    for t in range(4):
        st = sums_ref[0, t] + sums_ref[1, t]
        cnt = cnts_ref[0, t][:, 0:1] + cnts_ref[1, t][:, 0:1]
        parts.append(st / jnp.maximum(cnt, 1.0))
    parts.append(post_ref[...])
    x = jnp.concatenate(parts, axis=1)
    h = jnp.dot(x, w1_ref[...], preferred_element_type=jnp.float32) + b1_ref[...]
    h = jnp.where(h >= 0, h, 0.01 * h)
    o = jnp.dot(h, w2_ref[...], preferred_element_type=jnp.float32) + b2_ref[...]
    o = jnp.where(o >= 0, o, 0.01 * o)
    m = jnp.max(o, axis=-1, keepdims=True)
    e = jnp.exp(o - m)
    o_ref[...] = e / jnp.sum(e, axis=-1, keepdims=True)


def _mlp(sums, cnts, post_emb, W1, b1, W2, b2):
    BM = 256
    grid = (B // BM,)
    return pl.pallas_call(
        _mlp_body,
        grid=grid,
        in_specs=[
            pl.BlockSpec((NC, 4, BM, D), lambda i: (0, 0, i, 0)),
            pl.BlockSpec((NC, 4, BM, D), lambda i: (0, 0, i, 0)),
            pl.BlockSpec((BM, post_emb.shape[1]), lambda i: (i, 0)),
            pl.BlockSpec(W1.shape, lambda i: (0, 0)),
            pl.BlockSpec((1, b1.shape[0]), lambda i: (0, 0)),
            pl.BlockSpec(W2.shape, lambda i: (0, 0)),
            pl.BlockSpec((1, b2.shape[0]), lambda i: (0, 0)),
        ],
        out_specs=pl.BlockSpec((BM, 2), lambda i: (i, 0)),
        out_shape=jax.ShapeDtypeStruct((B, 2), jnp.float32),
    )(sums, cnts, post_emb, W1, b1.reshape(1, -1), W2, b2.reshape(1, -1))


def kernel(x_question, x_answer, x_comment, x_tag,
           batch_question, batch_answer, batch_comment, batch_tag,
           edge_index, post_emb, W1, b1, W2, b2):
    sums, cnts = _sc_pool(x_question, x_answer, x_comment, x_tag,
                          batch_question, batch_answer, batch_comment,
                          batch_tag)
    return _mlp(sums, cnts, post_emb, W1, b1, W2, b2)
